# baseline (speedup 1.0000x reference)
import jax
import jax.numpy as jnp
import numpy as np
from jax import lax
from jax.experimental import pallas as pl
from jax.experimental.pallas import tpu as pltpu
from jax.experimental.pallas import tpu_sc as plsc

H = 768
NH = 12
DH = H // NH
E = 6
R = 16
SCALE = 128.0 / 16.0
SBLK = 512
ABLK = 512

NC = 2
NS = 16
NW = NC * NS
L = 16
CH = 2048 // NW


def _dot(a, b):
    return jnp.dot(a, b, preferred_element_type=jnp.float32)


def _topk2_coef(scores):
    lane = jax.lax.broadcasted_iota(jnp.int32, scores.shape, 1)
    m1 = jnp.max(scores, axis=1, keepdims=True)
    i1 = jnp.min(jnp.where(scores == m1, lane, E), axis=1, keepdims=True)
    masked = jnp.where(lane == i1, -jnp.inf, scores)
    m2 = jnp.max(masked, axis=1, keepdims=True)
    i2 = jnp.min(jnp.where(masked == m2, lane, E), axis=1, keepdims=True)
    d = jnp.exp(m2 - m1)
    w1 = 1.0 / (1.0 + d)
    w2 = 1.0 - w1
    return jnp.where(lane == i1, w1, 0.0) + jnp.where(lane == i2, w2, 0.0)


def _expert_col(coef, e):
    lane = jax.lax.broadcasted_iota(jnp.int32, coef.shape, 1)
    return jnp.sum(jnp.where(lane == e, coef, 0.0), axis=1, keepdims=True)


def _gates_kernel(x_ref, gv_ref, go_ref, zv_ref, zo_ref):
    x32 = x_ref[...]
    zv_ref[...] = _dot(x32, gv_ref[...])
    zo_ref[...] = _dot(x32, go_ref[...])


def _gate_coef_sc(zv_hbm, zo_hbm, cv_hbm, co_hbm, zbuf, cbuf):
    wid = lax.axis_index("s") * NC + lax.axis_index("c")
    for z_hbm, c_hbm in ((zv_hbm, cv_hbm), (zo_hbm, co_hbm)):
        pltpu.sync_copy(z_hbm.at[wid], zbuf)
        for g in range(CH // L):
            sl = pl.ds(g * L, L)
            s = [1.0 / (1.0 + jnp.exp(-zbuf[e, sl])) for e in range(E)]
            m1 = s[0]
            for e in range(1, E):
                m1 = jnp.maximum(m1, s[e])
            i1 = jnp.full((L,), E, jnp.int32)
            for e in reversed(range(E)):
                i1 = jnp.where(s[e] == m1, e, i1)
            m2 = jnp.full((L,), -jnp.inf, jnp.float32)
            for e in range(E):
                m2 = jnp.maximum(m2, jnp.where(i1 == e, -jnp.inf, s[e]))
            i2 = jnp.full((L,), E, jnp.int32)
            for e in reversed(range(E)):
                i2 = jnp.where((s[e] == m2) & (i1 != e), e, i2)
            d = jnp.exp(m2 - m1)
            w1 = 1.0 / (1.0 + d)
            w2 = 1.0 - w1
            zero = jnp.zeros((L,), jnp.float32)
            for e in range(E):
                cbuf[e, sl] = jnp.where(i1 == e, w1, jnp.where(i2 == e, w2, zero))
        pltpu.sync_copy(cbuf, c_hbm.at[wid])


def _qk_kernel(x_ref, wqk_ref, acat_ref, bqk_ref, q_ref, k_ref, tv_ref):
    xb = x_ref[...].astype(jnp.bfloat16)
    t = _dot(xb, acat_ref[...])
    qk = _dot(xb, wqk_ref[...])
    qk = qk + SCALE * _dot(t[:, :2 * R].astype(jnp.bfloat16), bqk_ref[...])
    q_ref[...] = qk[:, :H].astype(jnp.bfloat16)
    k_ref[...] = qk[:, H:].astype(jnp.bfloat16)
    tv_ref[...] = t[:, 2 * R:]


def _vmoe_kernel(x_ref, tv_ref, cv_ref, wv_ref, bvs_ref, rep_ref, vt_ref):
    xb = x_ref[...].astype(jnp.bfloat16)
    cv = cv_ref[...]
    crep = _dot(cv, rep_ref[...])
    u = (tv_ref[...] * crep).astype(jnp.bfloat16)
    acc = SCALE * _dot(u, bvs_ref[...])
    for e in range(E):
        acc = acc + _expert_col(cv, e) * _dot(xb, wv_ref[e])
    vt_ref[...] = acc.astype(jnp.bfloat16).T


def _attn_kernel(q_ref, k_ref, vt_ref, mb_ref, ot_ref):
    q2 = q_ref[...] * jnp.bfloat16(0.125)
    k2 = k_ref[...]
    vt = vt_ref[...]
    bias = mb_ref[...]
    halves = []
    for i in range(2):
        st = jax.lax.dot_general(k2[:, DH * i:DH * (i + 1)],
                                 q2[:, DH * i:DH * (i + 1)],
                                 (((1,), (1,)), ((), ())),
                                 preferred_element_type=jnp.float32)
        st = st + bias
        mx = jnp.max(st, axis=0, keepdims=True)
        p = jnp.exp(st - mx)
        inv = 1.0 / jnp.sum(p, axis=0, keepdims=True)
        ct = _dot(vt[DH * i:DH * (i + 1), :], p.astype(jnp.bfloat16))
        halves.append(ct * inv)
    ot_ref[...] = jnp.concatenate(halves, axis=0).astype(jnp.bfloat16)


def _omoe_kernel(ct_ref, co_ref, wo_ref, aocat_ref, bos_ref, rep_ref, out_ref):
    cb = ct_ref[...].T
    co = co_ref[...]
    t = _dot(cb, aocat_ref[...])
    crep = _dot(co, rep_ref[...])
    u = (t * crep).astype(jnp.bfloat16)
    acc = SCALE * _dot(u, bos_ref[...])
    for e in range(E):
        acc = acc + _expert_col(co, e) * _dot(cb, wo_ref[e])
    out_ref[...] = acc


def _full(shape):
    return pl.BlockSpec(shape, lambda *_: (0,) * len(shape))


def kernel(hidden_states, attention_mask, Wq, Aq, Bq, Wk, Ak, Bk,
           gate_v_w, gate_o_w, Wv, Av, Bv, Wo, Ao, Bo):
    B, S, _ = hidden_states.shape
    x = hidden_states.reshape(S, H)
    f16 = jnp.bfloat16

    wqk = jnp.concatenate([Wq, Wk], axis=1).astype(f16)
    acat = jnp.concatenate(
        [Aq, Ak, Av.transpose(1, 0, 2).reshape(H, E * R)], axis=1).astype(f16)
    bqk = jnp.zeros((2 * R, 2 * H), jnp.float32)
    bqk = bqk.at[:R, :H].set(Bq).at[R:, H:].set(Bk).astype(f16)
    wv = Wv.astype(f16)
    bvs = Bv.reshape(E * R, H).astype(f16)
    wo = Wo.astype(f16)
    aocat = Ao.transpose(1, 0, 2).reshape(H, E * R).astype(f16)
    bos = Bo.reshape(E * R, H).astype(f16)
    rep = jnp.asarray(np.repeat(np.eye(E, dtype=np.float32), R, axis=1))
    mbias = ((1.0 - attention_mask) * -10000.0).reshape(S, 1)

    nblk = S // SBLK

    zv, zo = pl.pallas_call(
        _gates_kernel,
        grid=(nblk,),
        in_specs=[
            pl.BlockSpec((SBLK, H), lambda s: (s, 0)),
            _full((H, E)), _full((H, E)),
        ],
        out_specs=[
            pl.BlockSpec((SBLK, E), lambda s: (s, 0)),
            pl.BlockSpec((SBLK, E), lambda s: (s, 0)),
        ],
        out_shape=[
            jax.ShapeDtypeStruct((S, E), jnp.float32),
            jax.ShapeDtypeStruct((S, E), jnp.float32),
        ],
    )(x, gate_v_w, gate_o_w)

    zv_w = zv.reshape(NW, CH, E).transpose(0, 2, 1)
    zo_w = zo.reshape(NW, CH, E).transpose(0, 2, 1)
    gate_sc = pl.kernel(
        _gate_coef_sc,
        out_type=[
            jax.ShapeDtypeStruct((NW, E, CH), jnp.float32),
            jax.ShapeDtypeStruct((NW, E, CH), jnp.float32),
        ],
        scratch_types=[
            pltpu.VMEM((E, CH), jnp.float32),
            pltpu.VMEM((E, CH), jnp.float32),
        ],
        mesh=plsc.VectorSubcoreMesh(core_axis_name="c", subcore_axis_name="s"),
    )
    cv_w, co_w = gate_sc(zv_w, zo_w)
    cv = cv_w.transpose(0, 2, 1).reshape(S, E)
    co = co_w.transpose(0, 2, 1).reshape(S, E)

    q, k, tv = pl.pallas_call(
        _qk_kernel,
        grid=(nblk,),
        in_specs=[
            pl.BlockSpec((SBLK, H), lambda s: (s, 0)),
            _full((H, 2 * H)), _full((H, 2 * R + E * R)), _full((2 * R, 2 * H)),
        ],
        out_specs=[
            pl.BlockSpec((SBLK, H), lambda s: (s, 0)),
            pl.BlockSpec((SBLK, H), lambda s: (s, 0)),
            pl.BlockSpec((SBLK, E * R), lambda s: (s, 0)),
        ],
        out_shape=[
            jax.ShapeDtypeStruct((S, H), f16),
            jax.ShapeDtypeStruct((S, H), f16),
            jax.ShapeDtypeStruct((S, E * R), jnp.float32),
        ],
    )(x, wqk, acat, bqk)

    vt = pl.pallas_call(
        _vmoe_kernel,
        grid=(nblk,),
        in_specs=[
            pl.BlockSpec((SBLK, H), lambda s: (s, 0)),
            pl.BlockSpec((SBLK, E * R), lambda s: (s, 0)),
            pl.BlockSpec((SBLK, E), lambda s: (s, 0)),
            _full((E, H, H)), _full((E * R, H)), _full((E, E * R)),
        ],
        out_specs=pl.BlockSpec((H, SBLK), lambda s: (0, s)),
        out_shape=jax.ShapeDtypeStruct((H, S), f16),
    )(x, tv, cv, wv, bvs, rep)

    nab = S // ABLK
    ctx_t = pl.pallas_call(
        _attn_kernel,
        grid=(NH // 2, nab),
        in_specs=[
            pl.BlockSpec((ABLK, 2 * DH), lambda h, s: (s, h)),
            pl.BlockSpec((S, 2 * DH), lambda h, s: (0, h)),
            pl.BlockSpec((2 * DH, S), lambda h, s: (h, 0)),
            pl.BlockSpec((S, 1), lambda h, s: (0, 0)),
        ],
        out_specs=pl.BlockSpec((2 * DH, ABLK), lambda h, s: (h, s)),
        out_shape=jax.ShapeDtypeStruct((H, S), f16),
    )(q, k, vt, mbias)

    out = pl.pallas_call(
        _omoe_kernel,
        grid=(nblk,),
        in_specs=[
            pl.BlockSpec((H, SBLK), lambda s: (0, s)),
            pl.BlockSpec((SBLK, E), lambda s: (s, 0)),
            _full((E, H, H)), _full((H, E * R)), _full((E * R, H)),
            _full((E, E * R)),
        ],
        out_specs=pl.BlockSpec((SBLK, H), lambda s: (s, 0)),
        out_shape=jax.ShapeDtypeStruct((S, H), jnp.float32),
    )(ctx_t, co, wo, aocat, bos, rep)

    return out.reshape(B, S, H)


# speedup vs baseline: 1.0475x; 1.0475x over previous
import jax
import jax.numpy as jnp
import numpy as np
from jax import lax
from jax.experimental import pallas as pl
from jax.experimental.pallas import tpu as pltpu
from jax.experimental.pallas import tpu_sc as plsc

H = 768
NH = 12
DH = H // NH
E = 6
R = 16
SCALE = 128.0 / 16.0
SBLK = 512
ABLK = 512

NC = 2
NS = 16
NW = NC * NS
L = 16
CH = 128


def _dot(a, b):
    return jnp.dot(a, b, preferred_element_type=jnp.float32)


def _topk2_coef(scores):
    lane = jax.lax.broadcasted_iota(jnp.int32, scores.shape, 1)
    m1 = jnp.max(scores, axis=1, keepdims=True)
    i1 = jnp.min(jnp.where(scores == m1, lane, E), axis=1, keepdims=True)
    masked = jnp.where(lane == i1, -jnp.inf, scores)
    m2 = jnp.max(masked, axis=1, keepdims=True)
    i2 = jnp.min(jnp.where(masked == m2, lane, E), axis=1, keepdims=True)
    d = jnp.exp(m2 - m1)
    w1 = 1.0 / (1.0 + d)
    w2 = 1.0 - w1
    return jnp.where(lane == i1, w1, 0.0) + jnp.where(lane == i2, w2, 0.0)


def _expert_col(coef, e):
    lane = jax.lax.broadcasted_iota(jnp.int32, coef.shape, 1)
    return jnp.sum(jnp.where(lane == e, coef, 0.0), axis=1, keepdims=True)


def _gates_kernel(x_ref, gv_ref, go_ref, zv_ref, zo_ref):
    x32 = x_ref[...]
    zv_ref[...] = _dot(x32, gv_ref[...]).T
    zo_ref[...] = _dot(x32, go_ref[...]).T


def _gate_coef_sc(zv_hbm, zo_hbm, cv_hbm, co_hbm, zbuf, cbuf):
    wid = lax.axis_index("s") * NC + lax.axis_index("c")

    @pl.when(wid < 2048 // CH)
    def _():
        for z_hbm, c_hbm in ((zv_hbm, cv_hbm), (zo_hbm, co_hbm)):
            pltpu.sync_copy(z_hbm.at[:, pl.ds(wid * CH, CH)], zbuf)
            for g in range(CH // L):
                sl = pl.ds(g * L, L)
                s = [1.0 / (1.0 + jnp.exp(-zbuf[e, sl])) for e in range(E)]
                m1 = s[0]
                for e in range(1, E):
                    m1 = jnp.maximum(m1, s[e])
                i1 = jnp.full((L,), E, jnp.int32)
                for e in reversed(range(E)):
                    i1 = jnp.where(s[e] == m1, e, i1)
                m2 = jnp.full((L,), -jnp.inf, jnp.float32)
                for e in range(E):
                    m2 = jnp.maximum(m2, jnp.where(i1 == e, -jnp.inf, s[e]))
                i2 = jnp.full((L,), E, jnp.int32)
                for e in reversed(range(E)):
                    i2 = jnp.where((s[e] == m2) & (i1 != e), e, i2)
                d = jnp.exp(m2 - m1)
                w1 = 1.0 / (1.0 + d)
                w2 = 1.0 - w1
                zero = jnp.zeros((L,), jnp.float32)
                for e in range(E):
                    cbuf[e, sl] = jnp.where(i1 == e, w1,
                                            jnp.where(i2 == e, w2, zero))
            pltpu.sync_copy(cbuf, c_hbm.at[:, pl.ds(wid * CH, CH)])


def _qk_kernel(x_ref, wqk_ref, acat_ref, bqk_ref, q_ref, k_ref, tv_ref):
    xb = x_ref[...].astype(jnp.bfloat16)
    t = _dot(xb, acat_ref[...])
    qk = _dot(xb, wqk_ref[...])
    qk = qk + SCALE * _dot(t[:, :2 * R].astype(jnp.bfloat16), bqk_ref[...])
    q_ref[...] = qk[:, :H].astype(jnp.bfloat16)
    k_ref[...] = qk[:, H:].astype(jnp.bfloat16)
    tv_ref[...] = t[:, 2 * R:]


def _vmoe_kernel(x_ref, tv_ref, cv_ref, wv_ref, bvs_ref, rep_ref, vt_ref):
    xb = x_ref[...].astype(jnp.bfloat16)
    cv = cv_ref[...].T
    crep = _dot(cv, rep_ref[...])
    u = (tv_ref[...] * crep).astype(jnp.bfloat16)
    acc = SCALE * _dot(u, bvs_ref[...])
    for e in range(E):
        acc = acc + _expert_col(cv, e) * _dot(xb, wv_ref[e])
    vt_ref[...] = acc.astype(jnp.bfloat16).T


def _attn_kernel(q_ref, k_ref, vt_ref, mb_ref, ot_ref):
    q2 = q_ref[...] * jnp.bfloat16(0.125)
    k2 = k_ref[...]
    vt = vt_ref[...]
    bias = mb_ref[...]
    halves = []
    for i in range(2):
        st = jax.lax.dot_general(k2[:, DH * i:DH * (i + 1)],
                                 q2[:, DH * i:DH * (i + 1)],
                                 (((1,), (1,)), ((), ())),
                                 preferred_element_type=jnp.float32)
        st = st + bias
        mx = jnp.max(st, axis=0, keepdims=True)
        p = jnp.exp(st - mx)
        inv = 1.0 / jnp.sum(p, axis=0, keepdims=True)
        ct = _dot(vt[DH * i:DH * (i + 1), :], p.astype(jnp.bfloat16))
        halves.append(ct * inv)
    ot_ref[...] = jnp.concatenate(halves, axis=0).astype(jnp.bfloat16)


def _omoe_kernel(ct_ref, co_ref, wo_ref, aocat_ref, bos_ref, rep_ref, out_ref):
    cb = ct_ref[...].T
    co = co_ref[...].T
    t = _dot(cb, aocat_ref[...])
    crep = _dot(co, rep_ref[...])
    u = (t * crep).astype(jnp.bfloat16)
    acc = SCALE * _dot(u, bos_ref[...])
    for e in range(E):
        acc = acc + _expert_col(co, e) * _dot(cb, wo_ref[e])
    out_ref[...] = acc


def _full(shape):
    return pl.BlockSpec(shape, lambda *_: (0,) * len(shape))


def kernel(hidden_states, attention_mask, Wq, Aq, Bq, Wk, Ak, Bk,
           gate_v_w, gate_o_w, Wv, Av, Bv, Wo, Ao, Bo):
    B, S, _ = hidden_states.shape
    x = hidden_states.reshape(S, H)
    f16 = jnp.bfloat16

    wqk = jnp.concatenate([Wq, Wk], axis=1).astype(f16)
    acat = jnp.concatenate(
        [Aq, Ak, Av.transpose(1, 0, 2).reshape(H, E * R)], axis=1).astype(f16)
    bqk = jnp.zeros((2 * R, 2 * H), jnp.float32)
    bqk = bqk.at[:R, :H].set(Bq).at[R:, H:].set(Bk).astype(f16)
    wv = Wv.astype(f16)
    bvs = Bv.reshape(E * R, H).astype(f16)
    wo = Wo.astype(f16)
    aocat = Ao.transpose(1, 0, 2).reshape(H, E * R).astype(f16)
    bos = Bo.reshape(E * R, H).astype(f16)
    rep = jnp.asarray(np.repeat(np.eye(E, dtype=np.float32), R, axis=1))
    mbias = ((1.0 - attention_mask) * -10000.0).reshape(S, 1)

    nblk = S // SBLK

    zv, zo = pl.pallas_call(
        _gates_kernel,
        grid=(nblk,),
        in_specs=[
            pl.BlockSpec((SBLK, H), lambda s: (s, 0)),
            _full((H, E)), _full((H, E)),
        ],
        out_specs=[
            pl.BlockSpec((E, SBLK), lambda s: (0, s)),
            pl.BlockSpec((E, SBLK), lambda s: (0, s)),
        ],
        out_shape=[
            jax.ShapeDtypeStruct((E, S), jnp.float32),
            jax.ShapeDtypeStruct((E, S), jnp.float32),
        ],
    )(x, gate_v_w, gate_o_w)

    gate_sc = pl.kernel(
        _gate_coef_sc,
        out_type=[
            jax.ShapeDtypeStruct((E, S), jnp.float32),
            jax.ShapeDtypeStruct((E, S), jnp.float32),
        ],
        scratch_types=[
            pltpu.VMEM((E, CH), jnp.float32),
            pltpu.VMEM((E, CH), jnp.float32),
        ],
        mesh=plsc.VectorSubcoreMesh(core_axis_name="c", subcore_axis_name="s"),
    )
    cv, co = gate_sc(zv, zo)

    q, k, tv = pl.pallas_call(
        _qk_kernel,
        grid=(nblk,),
        in_specs=[
            pl.BlockSpec((SBLK, H), lambda s: (s, 0)),
            _full((H, 2 * H)), _full((H, 2 * R + E * R)), _full((2 * R, 2 * H)),
        ],
        out_specs=[
            pl.BlockSpec((SBLK, H), lambda s: (s, 0)),
            pl.BlockSpec((SBLK, H), lambda s: (s, 0)),
            pl.BlockSpec((SBLK, E * R), lambda s: (s, 0)),
        ],
        out_shape=[
            jax.ShapeDtypeStruct((S, H), f16),
            jax.ShapeDtypeStruct((S, H), f16),
            jax.ShapeDtypeStruct((S, E * R), jnp.float32),
        ],
    )(x, wqk, acat, bqk)

    vt = pl.pallas_call(
        _vmoe_kernel,
        grid=(nblk,),
        in_specs=[
            pl.BlockSpec((SBLK, H), lambda s: (s, 0)),
            pl.BlockSpec((SBLK, E * R), lambda s: (s, 0)),
            pl.BlockSpec((E, SBLK), lambda s: (0, s)),
            _full((E, H, H)), _full((E * R, H)), _full((E, E * R)),
        ],
        out_specs=pl.BlockSpec((H, SBLK), lambda s: (0, s)),
        out_shape=jax.ShapeDtypeStruct((H, S), f16),
    )(x, tv, cv, wv, bvs, rep)

    nab = S // ABLK
    ctx_t = pl.pallas_call(
        _attn_kernel,
        grid=(NH // 2, nab),
        in_specs=[
            pl.BlockSpec((ABLK, 2 * DH), lambda h, s: (s, h)),
            pl.BlockSpec((S, 2 * DH), lambda h, s: (0, h)),
            pl.BlockSpec((2 * DH, S), lambda h, s: (h, 0)),
            pl.BlockSpec((S, 1), lambda h, s: (0, 0)),
        ],
        out_specs=pl.BlockSpec((2 * DH, ABLK), lambda h, s: (h, s)),
        out_shape=jax.ShapeDtypeStruct((H, S), f16),
    )(q, k, vt, mbias)

    out = pl.pallas_call(
        _omoe_kernel,
        grid=(nblk,),
        in_specs=[
            pl.BlockSpec((H, SBLK), lambda s: (0, s)),
            pl.BlockSpec((E, SBLK), lambda s: (0, s)),
            _full((E, H, H)), _full((H, E * R)), _full((E * R, H)),
            _full((E, E * R)),
        ],
        out_specs=pl.BlockSpec((SBLK, H), lambda s: (s, 0)),
        out_shape=jax.ShapeDtypeStruct((S, H), jnp.float32),
    )(ctx_t, co, wo, aocat, bos, rep)

    return out.reshape(B, S, H)


# speedup vs baseline: 1.0680x; 1.0196x over previous
import jax
import jax.numpy as jnp
import numpy as np
from jax import lax
from jax.experimental import pallas as pl
from jax.experimental.pallas import tpu as pltpu
from jax.experimental.pallas import tpu_sc as plsc

H = 768
NH = 12
DH = H // NH
E = 6
R = 16
SCALE = 128.0 / 16.0
SBLK = 512
ABLK = 512

NC = 2
NS = 16
NW = NC * NS
L = 16
CH = 128


def _dot(a, b):
    return jnp.dot(a, b, preferred_element_type=jnp.float32)


def _topk2_coef(scores):
    lane = jax.lax.broadcasted_iota(jnp.int32, scores.shape, 1)
    m1 = jnp.max(scores, axis=1, keepdims=True)
    i1 = jnp.min(jnp.where(scores == m1, lane, E), axis=1, keepdims=True)
    masked = jnp.where(lane == i1, -jnp.inf, scores)
    m2 = jnp.max(masked, axis=1, keepdims=True)
    i2 = jnp.min(jnp.where(masked == m2, lane, E), axis=1, keepdims=True)
    d = jnp.exp(m2 - m1)
    w1 = 1.0 / (1.0 + d)
    w2 = 1.0 - w1
    return jnp.where(lane == i1, w1, 0.0) + jnp.where(lane == i2, w2, 0.0)


def _expert_col(coef, e):
    lane = jax.lax.broadcasted_iota(jnp.int32, coef.shape, 1)
    return jnp.sum(jnp.where(lane == e, coef, 0.0), axis=1, keepdims=True)


def _qkg_kernel(x_ref, gv_ref, go_ref, wqk_ref, acat_ref, bqk_ref,
                zv_ref, zo_ref, q_ref, k_ref, tv_ref):
    x32 = x_ref[...]
    zv_ref[...] = _dot(x32, gv_ref[...]).T
    zo_ref[...] = _dot(x32, go_ref[...]).T
    xb = x32.astype(jnp.bfloat16)
    t = _dot(xb, acat_ref[...])
    qk = _dot(xb, wqk_ref[...])
    qk = qk + SCALE * _dot(t[:, :2 * R].astype(jnp.bfloat16), bqk_ref[...])
    q_ref[...] = qk[:, :H].astype(jnp.bfloat16)
    k_ref[...] = qk[:, H:].astype(jnp.bfloat16)
    tv_ref[...] = t[:, 2 * R:]


def _gate_coef_sc(zv_hbm, zo_hbm, cv_hbm, co_hbm, zbuf, cbuf):
    wid = lax.axis_index("s") * NC + lax.axis_index("c")

    @pl.when(wid < 2048 // CH)
    def _():
        for z_hbm, c_hbm in ((zv_hbm, cv_hbm), (zo_hbm, co_hbm)):
            pltpu.sync_copy(z_hbm.at[:, pl.ds(wid * CH, CH)], zbuf)
            for g in range(CH // L):
                sl = pl.ds(g * L, L)
                s = [1.0 / (1.0 + jnp.exp(-zbuf[e, sl])) for e in range(E)]
                m1 = s[0]
                for e in range(1, E):
                    m1 = jnp.maximum(m1, s[e])
                i1 = jnp.full((L,), E, jnp.int32)
                for e in reversed(range(E)):
                    i1 = jnp.where(s[e] == m1, e, i1)
                m2 = jnp.full((L,), -jnp.inf, jnp.float32)
                for e in range(E):
                    m2 = jnp.maximum(m2, jnp.where(i1 == e, -jnp.inf, s[e]))
                i2 = jnp.full((L,), E, jnp.int32)
                for e in reversed(range(E)):
                    i2 = jnp.where((s[e] == m2) & (i1 != e), e, i2)
                d = jnp.exp(m2 - m1)
                w1 = 1.0 / (1.0 + d)
                w2 = 1.0 - w1
                zero = jnp.zeros((L,), jnp.float32)
                for e in range(E):
                    cbuf[e, sl] = jnp.where(i1 == e, w1,
                                            jnp.where(i2 == e, w2, zero))
            pltpu.sync_copy(cbuf, c_hbm.at[:, pl.ds(wid * CH, CH)])


def _vmoe_kernel(x_ref, tv_ref, cv_ref, wv_ref, bvs_ref, rep_ref, vt_ref):
    xb = x_ref[...].astype(jnp.bfloat16)
    cv = cv_ref[...].T
    crep = _dot(cv, rep_ref[...])
    u = (tv_ref[...] * crep).astype(jnp.bfloat16)
    acc = SCALE * _dot(u, bvs_ref[...])
    for e in range(E):
        acc = acc + _expert_col(cv, e) * _dot(xb, wv_ref[e])
    vt_ref[...] = acc.astype(jnp.bfloat16).T


def _attn_kernel(q_ref, k_ref, vt_ref, mb_ref, ot_ref):
    q2 = q_ref[...] * jnp.bfloat16(0.125)
    k2 = k_ref[...]
    vt = vt_ref[...]
    bias = mb_ref[...]
    halves = []
    for i in range(2):
        st = jax.lax.dot_general(k2[:, DH * i:DH * (i + 1)],
                                 q2[:, DH * i:DH * (i + 1)],
                                 (((1,), (1,)), ((), ())),
                                 preferred_element_type=jnp.float32)
        st = st + bias
        mx = jnp.max(st, axis=0, keepdims=True)
        p = jnp.exp(st - mx)
        inv = 1.0 / jnp.sum(p, axis=0, keepdims=True)
        ct = _dot(vt[DH * i:DH * (i + 1), :], p.astype(jnp.bfloat16))
        halves.append(ct * inv)
    ot_ref[...] = jnp.concatenate(halves, axis=0).astype(jnp.bfloat16)


def _omoe_kernel(ct_ref, co_ref, wo_ref, aocat_ref, bos_ref, rep_ref, out_ref):
    cb = ct_ref[...].T
    co = co_ref[...].T
    t = _dot(cb, aocat_ref[...])
    crep = _dot(co, rep_ref[...])
    u = (t * crep).astype(jnp.bfloat16)
    acc = SCALE * _dot(u, bos_ref[...])
    for e in range(E):
        acc = acc + _expert_col(co, e) * _dot(cb, wo_ref[e])
    out_ref[...] = acc


def _full(shape):
    return pl.BlockSpec(shape, lambda *_: (0,) * len(shape))


def kernel(hidden_states, attention_mask, Wq, Aq, Bq, Wk, Ak, Bk,
           gate_v_w, gate_o_w, Wv, Av, Bv, Wo, Ao, Bo):
    B, S, _ = hidden_states.shape
    x = hidden_states.reshape(S, H)
    f16 = jnp.bfloat16

    wqk = jnp.concatenate([Wq, Wk], axis=1).astype(f16)
    acat = jnp.concatenate(
        [Aq, Ak, Av.transpose(1, 0, 2).reshape(H, E * R)], axis=1).astype(f16)
    bqk = jnp.zeros((2 * R, 2 * H), jnp.float32)
    bqk = bqk.at[:R, :H].set(Bq).at[R:, H:].set(Bk).astype(f16)
    wv = Wv.astype(f16)
    bvs = Bv.reshape(E * R, H).astype(f16)
    wo = Wo.astype(f16)
    aocat = Ao.transpose(1, 0, 2).reshape(H, E * R).astype(f16)
    bos = Bo.reshape(E * R, H).astype(f16)
    rep = jnp.asarray(np.repeat(np.eye(E, dtype=np.float32), R, axis=1))
    mbias = ((1.0 - attention_mask) * -10000.0).reshape(S, 1)

    nblk = S // SBLK

    zv, zo, q, k, tv = pl.pallas_call(
        _qkg_kernel,
        grid=(nblk,),
        in_specs=[
            pl.BlockSpec((SBLK, H), lambda s: (s, 0)),
            _full((H, E)), _full((H, E)),
            _full((H, 2 * H)), _full((H, 2 * R + E * R)), _full((2 * R, 2 * H)),
        ],
        out_specs=[
            pl.BlockSpec((E, SBLK), lambda s: (0, s)),
            pl.BlockSpec((E, SBLK), lambda s: (0, s)),
            pl.BlockSpec((SBLK, H), lambda s: (s, 0)),
            pl.BlockSpec((SBLK, H), lambda s: (s, 0)),
            pl.BlockSpec((SBLK, E * R), lambda s: (s, 0)),
        ],
        out_shape=[
            jax.ShapeDtypeStruct((E, S), jnp.float32),
            jax.ShapeDtypeStruct((E, S), jnp.float32),
            jax.ShapeDtypeStruct((S, H), f16),
            jax.ShapeDtypeStruct((S, H), f16),
            jax.ShapeDtypeStruct((S, E * R), jnp.float32),
        ],
    )(x, gate_v_w, gate_o_w, wqk, acat, bqk)

    gate_sc = pl.kernel(
        _gate_coef_sc,
        out_type=[
            jax.ShapeDtypeStruct((E, S), jnp.float32),
            jax.ShapeDtypeStruct((E, S), jnp.float32),
        ],
        scratch_types=[
            pltpu.VMEM((E, CH), jnp.float32),
            pltpu.VMEM((E, CH), jnp.float32),
        ],
        mesh=plsc.VectorSubcoreMesh(core_axis_name="c", subcore_axis_name="s"),
    )
    cv, co = gate_sc(zv, zo)

    vt = pl.pallas_call(
        _vmoe_kernel,
        grid=(nblk,),
        in_specs=[
            pl.BlockSpec((SBLK, H), lambda s: (s, 0)),
            pl.BlockSpec((SBLK, E * R), lambda s: (s, 0)),
            pl.BlockSpec((E, SBLK), lambda s: (0, s)),
            _full((E, H, H)), _full((E * R, H)), _full((E, E * R)),
        ],
        out_specs=pl.BlockSpec((H, SBLK), lambda s: (0, s)),
        out_shape=jax.ShapeDtypeStruct((H, S), f16),
    )(x, tv, cv, wv, bvs, rep)

    nab = S // ABLK
    ctx_t = pl.pallas_call(
        _attn_kernel,
        grid=(NH // 2, nab),
        in_specs=[
            pl.BlockSpec((ABLK, 2 * DH), lambda h, s: (s, h)),
            pl.BlockSpec((S, 2 * DH), lambda h, s: (0, h)),
            pl.BlockSpec((2 * DH, S), lambda h, s: (h, 0)),
            pl.BlockSpec((S, 1), lambda h, s: (0, 0)),
        ],
        out_specs=pl.BlockSpec((2 * DH, ABLK), lambda h, s: (h, s)),
        out_shape=jax.ShapeDtypeStruct((H, S), f16),
    )(q, k, vt, mbias)

    out = pl.pallas_call(
        _omoe_kernel,
        grid=(nblk,),
        in_specs=[
            pl.BlockSpec((H, SBLK), lambda s: (0, s)),
            pl.BlockSpec((E, SBLK), lambda s: (0, s)),
            _full((E, H, H)), _full((H, E * R)), _full((E * R, H)),
            _full((E, E * R)),
        ],
        out_specs=pl.BlockSpec((SBLK, H), lambda s: (s, 0)),
        out_shape=jax.ShapeDtypeStruct((S, H), jnp.float32),
    )(ctx_t, co, wo, aocat, bos, rep)

    return out.reshape(B, S, H)


# speedup vs baseline: 1.0734x; 1.0050x over previous
import jax
import jax.numpy as jnp
import numpy as np
from jax import lax
from jax.experimental import pallas as pl
from jax.experimental.pallas import tpu as pltpu
from jax.experimental.pallas import tpu_sc as plsc

H = 768
NH = 12
DH = H // NH
E = 6
R = 16
SCALE = 128.0 / 16.0
SBLK = 512
ABLK = 512

NC = 2
NS = 16
NW = NC * NS
L = 16
CH = 128


def _dot(a, b):
    return jnp.dot(a, b, preferred_element_type=jnp.float32)


def _topk2_coef(scores):
    lane = jax.lax.broadcasted_iota(jnp.int32, scores.shape, 1)
    m1 = jnp.max(scores, axis=1, keepdims=True)
    i1 = jnp.min(jnp.where(scores == m1, lane, E), axis=1, keepdims=True)
    masked = jnp.where(lane == i1, -jnp.inf, scores)
    m2 = jnp.max(masked, axis=1, keepdims=True)
    i2 = jnp.min(jnp.where(masked == m2, lane, E), axis=1, keepdims=True)
    d = jnp.exp(m2 - m1)
    w1 = 1.0 / (1.0 + d)
    w2 = 1.0 - w1
    return jnp.where(lane == i1, w1, 0.0) + jnp.where(lane == i2, w2, 0.0)


def _expert_col(coef, e):
    lane = jax.lax.broadcasted_iota(jnp.int32, coef.shape, 1)
    return jnp.sum(jnp.where(lane == e, coef, 0.0), axis=1, keepdims=True)


def _qkg_kernel(x_ref, gv_ref, go_ref, wqk_ref, acat_ref, bqk_ref,
                cv_ref, zo_ref, q_ref, k_ref, tv_ref):
    x32 = x_ref[...]
    cv_ref[...] = _topk2_coef(jax.nn.sigmoid(_dot(x32, gv_ref[...])))
    zo_ref[...] = _dot(x32, go_ref[...]).T
    xb = x32.astype(jnp.bfloat16)
    t = _dot(xb, acat_ref[...])
    qk = _dot(xb, wqk_ref[...])
    qk = qk + SCALE * _dot(t[:, :2 * R].astype(jnp.bfloat16), bqk_ref[...])
    q_ref[...] = qk[:, :H].astype(jnp.bfloat16)
    k_ref[...] = qk[:, H:].astype(jnp.bfloat16)
    tv_ref[...] = t[:, 2 * R:]


def _gate_coef_sc(zo_hbm, co_hbm, zbuf, cbuf):
    wid = lax.axis_index("s") * NC + lax.axis_index("c")

    @pl.when(wid < 2048 // CH)
    def _():
        for z_hbm, c_hbm in ((zo_hbm, co_hbm),):
            pltpu.sync_copy(z_hbm.at[:, pl.ds(wid * CH, CH)], zbuf)
            for g in range(CH // L):
                sl = pl.ds(g * L, L)
                s = [1.0 / (1.0 + jnp.exp(-zbuf[e, sl])) for e in range(E)]
                m1 = s[0]
                for e in range(1, E):
                    m1 = jnp.maximum(m1, s[e])
                i1 = jnp.full((L,), E, jnp.int32)
                for e in reversed(range(E)):
                    i1 = jnp.where(s[e] == m1, e, i1)
                m2 = jnp.full((L,), -jnp.inf, jnp.float32)
                for e in range(E):
                    m2 = jnp.maximum(m2, jnp.where(i1 == e, -jnp.inf, s[e]))
                i2 = jnp.full((L,), E, jnp.int32)
                for e in reversed(range(E)):
                    i2 = jnp.where((s[e] == m2) & (i1 != e), e, i2)
                d = jnp.exp(m2 - m1)
                w1 = 1.0 / (1.0 + d)
                w2 = 1.0 - w1
                zero = jnp.zeros((L,), jnp.float32)
                for e in range(E):
                    cbuf[e, sl] = jnp.where(i1 == e, w1,
                                            jnp.where(i2 == e, w2, zero))
            pltpu.sync_copy(cbuf, c_hbm.at[:, pl.ds(wid * CH, CH)])


def _vmoe_kernel(x_ref, tv_ref, cv_ref, wv_ref, bvs_ref, rep_ref, vt_ref):
    xb = x_ref[...].astype(jnp.bfloat16)
    cv = cv_ref[...]
    crep = _dot(cv, rep_ref[...])
    u = (tv_ref[...] * crep).astype(jnp.bfloat16)
    acc = SCALE * _dot(u, bvs_ref[...])
    for e in range(E):
        acc = acc + _expert_col(cv, e) * _dot(xb, wv_ref[e])
    vt_ref[...] = acc.astype(jnp.bfloat16).T


def _attn_kernel(q_ref, k_ref, vt_ref, mb_ref, ot_ref):
    q2 = q_ref[...] * jnp.bfloat16(0.125)
    k2 = k_ref[...]
    vt = vt_ref[...]
    bias = mb_ref[...]
    halves = []
    for i in range(2):
        st = jax.lax.dot_general(k2[:, DH * i:DH * (i + 1)],
                                 q2[:, DH * i:DH * (i + 1)],
                                 (((1,), (1,)), ((), ())),
                                 preferred_element_type=jnp.float32)
        st = st + bias
        mx = jnp.max(st, axis=0, keepdims=True)
        p = jnp.exp(st - mx)
        inv = 1.0 / jnp.sum(p, axis=0, keepdims=True)
        ct = _dot(vt[DH * i:DH * (i + 1), :], p.astype(jnp.bfloat16))
        halves.append(ct * inv)
    ot_ref[...] = jnp.concatenate(halves, axis=0).astype(jnp.bfloat16)


def _omoe_kernel(ct_ref, co_ref, wo_ref, aocat_ref, bos_ref, rep_ref, out_ref):
    cb = ct_ref[...].T
    co = co_ref[...].T
    t = _dot(cb, aocat_ref[...])
    crep = _dot(co, rep_ref[...])
    u = (t * crep).astype(jnp.bfloat16)
    acc = SCALE * _dot(u, bos_ref[...])
    for e in range(E):
        acc = acc + _expert_col(co, e) * _dot(cb, wo_ref[e])
    out_ref[...] = acc


def _full(shape):
    return pl.BlockSpec(shape, lambda *_: (0,) * len(shape))


def kernel(hidden_states, attention_mask, Wq, Aq, Bq, Wk, Ak, Bk,
           gate_v_w, gate_o_w, Wv, Av, Bv, Wo, Ao, Bo):
    B, S, _ = hidden_states.shape
    x = hidden_states.reshape(S, H)
    f16 = jnp.bfloat16

    wqk = jnp.concatenate([Wq, Wk], axis=1).astype(f16)
    acat = jnp.concatenate(
        [Aq, Ak, Av.transpose(1, 0, 2).reshape(H, E * R)], axis=1).astype(f16)
    bqk = jnp.zeros((2 * R, 2 * H), jnp.float32)
    bqk = bqk.at[:R, :H].set(Bq).at[R:, H:].set(Bk).astype(f16)
    wv = Wv.astype(f16)
    bvs = Bv.reshape(E * R, H).astype(f16)
    wo = Wo.astype(f16)
    aocat = Ao.transpose(1, 0, 2).reshape(H, E * R).astype(f16)
    bos = Bo.reshape(E * R, H).astype(f16)
    rep = jnp.asarray(np.repeat(np.eye(E, dtype=np.float32), R, axis=1))
    mbias = ((1.0 - attention_mask) * -10000.0).reshape(S, 1)

    nblk = S // SBLK

    cv, zo, q, k, tv = pl.pallas_call(
        _qkg_kernel,
        grid=(nblk,),
        in_specs=[
            pl.BlockSpec((SBLK, H), lambda s: (s, 0)),
            _full((H, E)), _full((H, E)),
            _full((H, 2 * H)), _full((H, 2 * R + E * R)), _full((2 * R, 2 * H)),
        ],
        out_specs=[
            pl.BlockSpec((SBLK, E), lambda s: (s, 0)),
            pl.BlockSpec((E, SBLK), lambda s: (0, s)),
            pl.BlockSpec((SBLK, H), lambda s: (s, 0)),
            pl.BlockSpec((SBLK, H), lambda s: (s, 0)),
            pl.BlockSpec((SBLK, E * R), lambda s: (s, 0)),
        ],
        out_shape=[
            jax.ShapeDtypeStruct((S, E), jnp.float32),
            jax.ShapeDtypeStruct((E, S), jnp.float32),
            jax.ShapeDtypeStruct((S, H), f16),
            jax.ShapeDtypeStruct((S, H), f16),
            jax.ShapeDtypeStruct((S, E * R), jnp.float32),
        ],
    )(x, gate_v_w, gate_o_w, wqk, acat, bqk)

    gate_sc = pl.kernel(
        _gate_coef_sc,
        out_type=jax.ShapeDtypeStruct((E, S), jnp.float32),
        scratch_types=[
            pltpu.VMEM((E, CH), jnp.float32),
            pltpu.VMEM((E, CH), jnp.float32),
        ],
        mesh=plsc.VectorSubcoreMesh(core_axis_name="c", subcore_axis_name="s"),
    )
    co = gate_sc(zo)

    vt = pl.pallas_call(
        _vmoe_kernel,
        grid=(nblk,),
        in_specs=[
            pl.BlockSpec((SBLK, H), lambda s: (s, 0)),
            pl.BlockSpec((SBLK, E * R), lambda s: (s, 0)),
            pl.BlockSpec((SBLK, E), lambda s: (s, 0)),
            _full((E, H, H)), _full((E * R, H)), _full((E, E * R)),
        ],
        out_specs=pl.BlockSpec((H, SBLK), lambda s: (0, s)),
        out_shape=jax.ShapeDtypeStruct((H, S), f16),
    )(x, tv, cv, wv, bvs, rep)

    nab = S // ABLK
    ctx_t = pl.pallas_call(
        _attn_kernel,
        grid=(NH // 2, nab),
        in_specs=[
            pl.BlockSpec((ABLK, 2 * DH), lambda h, s: (s, h)),
            pl.BlockSpec((S, 2 * DH), lambda h, s: (0, h)),
            pl.BlockSpec((2 * DH, S), lambda h, s: (h, 0)),
            pl.BlockSpec((S, 1), lambda h, s: (0, 0)),
        ],
        out_specs=pl.BlockSpec((2 * DH, ABLK), lambda h, s: (h, s)),
        out_shape=jax.ShapeDtypeStruct((H, S), f16),
    )(q, k, vt, mbias)

    out = pl.pallas_call(
        _omoe_kernel,
        grid=(nblk,),
        in_specs=[
            pl.BlockSpec((H, SBLK), lambda s: (0, s)),
            pl.BlockSpec((E, SBLK), lambda s: (0, s)),
            _full((E, H, H)), _full((H, E * R)), _full((E * R, H)),
            _full((E, E * R)),
        ],
        out_specs=pl.BlockSpec((SBLK, H), lambda s: (s, 0)),
        out_shape=jax.ShapeDtypeStruct((S, H), jnp.float32),
    )(ctx_t, co, wo, aocat, bos, rep)

    return out.reshape(B, S, H)


# speedup vs baseline: 1.0893x; 1.0148x over previous
import jax
import jax.numpy as jnp
import numpy as np
from jax import lax
from jax.experimental import pallas as pl
from jax.experimental.pallas import tpu as pltpu
from jax.experimental.pallas import tpu_sc as plsc

H = 768
NH = 12
DH = H // NH
E = 6
R = 16
SCALE = 128.0 / 16.0
SBLK = 512
ABLK = 512

NC = 2
NS = 16
NW = NC * NS
L = 16
CH = 128


def _dot(a, b):
    return jnp.dot(a, b, preferred_element_type=jnp.float32)


def _topk2_coef(scores):
    lane = jax.lax.broadcasted_iota(jnp.int32, scores.shape, 1)
    m1 = jnp.max(scores, axis=1, keepdims=True)
    i1 = jnp.min(jnp.where(scores == m1, lane, E), axis=1, keepdims=True)
    masked = jnp.where(lane == i1, -jnp.inf, scores)
    m2 = jnp.max(masked, axis=1, keepdims=True)
    i2 = jnp.min(jnp.where(masked == m2, lane, E), axis=1, keepdims=True)
    d = jnp.exp(m2 - m1)
    w1 = 1.0 / (1.0 + d)
    w2 = 1.0 - w1
    return jnp.where(lane == i1, w1, 0.0) + jnp.where(lane == i2, w2, 0.0)


def _expert_col(coef, e):
    lane = jax.lax.broadcasted_iota(jnp.int32, coef.shape, 1)
    return jnp.sum(jnp.where(lane == e, coef, 0.0), axis=1, keepdims=True)


def _pre_kernel(x_ref, gv_ref, go_ref, wqk_ref, acat_ref, bqk_ref,
                wv_ref, bvs_ref, rep_ref,
                zo_ref, q_ref, k_ref, vt_ref):
    x32 = x_ref[...]
    cv = _topk2_coef(jax.nn.sigmoid(_dot(x32, gv_ref[...])))
    zo_ref[...] = _dot(x32, go_ref[...]).T
    xb = x32.astype(jnp.bfloat16)
    t = _dot(xb, acat_ref[...])
    qk = _dot(xb, wqk_ref[...])
    qk = qk + SCALE * _dot(t[:, :2 * R].astype(jnp.bfloat16), bqk_ref[...])
    q_ref[...] = qk[:, :H].astype(jnp.bfloat16)
    k_ref[...] = qk[:, H:].astype(jnp.bfloat16)
    crep = _dot(cv, rep_ref[...])
    u = (t[:, 2 * R:] * crep).astype(jnp.bfloat16)
    acc = SCALE * _dot(u, bvs_ref[...])
    for e in range(E):
        acc = acc + _expert_col(cv, e) * _dot(xb, wv_ref[e])
    vt_ref[...] = acc.astype(jnp.bfloat16).T


def _gate_coef_sc(zo_hbm, co_hbm, zbuf, cbuf):
    wid = lax.axis_index("s") * NC + lax.axis_index("c")

    @pl.when(wid < 2048 // CH)
    def _():
        for z_hbm, c_hbm in ((zo_hbm, co_hbm),):
            pltpu.sync_copy(z_hbm.at[:, pl.ds(wid * CH, CH)], zbuf)
            for g in range(CH // L):
                sl = pl.ds(g * L, L)
                s = [1.0 / (1.0 + jnp.exp(-zbuf[e, sl])) for e in range(E)]
                m1 = s[0]
                for e in range(1, E):
                    m1 = jnp.maximum(m1, s[e])
                i1 = jnp.full((L,), E, jnp.int32)
                for e in reversed(range(E)):
                    i1 = jnp.where(s[e] == m1, e, i1)
                m2 = jnp.full((L,), -jnp.inf, jnp.float32)
                for e in range(E):
                    m2 = jnp.maximum(m2, jnp.where(i1 == e, -jnp.inf, s[e]))
                i2 = jnp.full((L,), E, jnp.int32)
                for e in reversed(range(E)):
                    i2 = jnp.where((s[e] == m2) & (i1 != e), e, i2)
                d = jnp.exp(m2 - m1)
                w1 = 1.0 / (1.0 + d)
                w2 = 1.0 - w1
                zero = jnp.zeros((L,), jnp.float32)
                for e in range(E):
                    cbuf[e, sl] = jnp.where(i1 == e, w1,
                                            jnp.where(i2 == e, w2, zero))
            pltpu.sync_copy(cbuf, c_hbm.at[:, pl.ds(wid * CH, CH)])


def _attn_kernel(q_ref, k_ref, vt_ref, mb_ref, ot_ref):
    q2 = q_ref[...] * jnp.bfloat16(0.125)
    k2 = k_ref[...]
    vt = vt_ref[...]
    bias = mb_ref[...]
    halves = []
    for i in range(2):
        st = jax.lax.dot_general(k2[:, DH * i:DH * (i + 1)],
                                 q2[:, DH * i:DH * (i + 1)],
                                 (((1,), (1,)), ((), ())),
                                 preferred_element_type=jnp.float32)
        st = st + bias
        mx = jnp.max(st, axis=0, keepdims=True)
        p = jnp.exp(st - mx)
        inv = 1.0 / jnp.sum(p, axis=0, keepdims=True)
        ct = _dot(vt[DH * i:DH * (i + 1), :], p.astype(jnp.bfloat16))
        halves.append(ct * inv)
    ot_ref[...] = jnp.concatenate(halves, axis=0).astype(jnp.bfloat16)


def _omoe_kernel(ct_ref, co_ref, wo_ref, aocat_ref, bos_ref, rep_ref, out_ref):
    cb = ct_ref[...].T
    co = co_ref[...].T
    t = _dot(cb, aocat_ref[...])
    crep = _dot(co, rep_ref[...])
    u = (t * crep).astype(jnp.bfloat16)
    acc = SCALE * _dot(u, bos_ref[...])
    for e in range(E):
        acc = acc + _expert_col(co, e) * _dot(cb, wo_ref[e])
    out_ref[...] = acc


def _full(shape):
    return pl.BlockSpec(shape, lambda *_: (0,) * len(shape))


def kernel(hidden_states, attention_mask, Wq, Aq, Bq, Wk, Ak, Bk,
           gate_v_w, gate_o_w, Wv, Av, Bv, Wo, Ao, Bo):
    B, S, _ = hidden_states.shape
    x = hidden_states.reshape(S, H)
    f16 = jnp.bfloat16

    wqk = jnp.concatenate([Wq, Wk], axis=1).astype(f16)
    acat = jnp.concatenate(
        [Aq, Ak, Av.transpose(1, 0, 2).reshape(H, E * R)], axis=1).astype(f16)
    bqk = jnp.zeros((2 * R, 2 * H), jnp.float32)
    bqk = bqk.at[:R, :H].set(Bq).at[R:, H:].set(Bk).astype(f16)
    wv = Wv.astype(f16)
    bvs = Bv.reshape(E * R, H).astype(f16)
    wo = Wo.astype(f16)
    aocat = Ao.transpose(1, 0, 2).reshape(H, E * R).astype(f16)
    bos = Bo.reshape(E * R, H).astype(f16)
    rep = jnp.asarray(np.repeat(np.eye(E, dtype=np.float32), R, axis=1))
    mbias = ((1.0 - attention_mask) * -10000.0).reshape(S, 1)

    nblk = S // SBLK

    zo, q, k, vt = pl.pallas_call(
        _pre_kernel,
        grid=(nblk,),
        in_specs=[
            pl.BlockSpec((SBLK, H), lambda s: (s, 0)),
            _full((H, E)), _full((H, E)),
            _full((H, 2 * H)), _full((H, 2 * R + E * R)), _full((2 * R, 2 * H)),
            _full((E, H, H)), _full((E * R, H)), _full((E, E * R)),
        ],
        out_specs=[
            pl.BlockSpec((E, SBLK), lambda s: (0, s)),
            pl.BlockSpec((SBLK, H), lambda s: (s, 0)),
            pl.BlockSpec((SBLK, H), lambda s: (s, 0)),
            pl.BlockSpec((H, SBLK), lambda s: (0, s)),
        ],
        out_shape=[
            jax.ShapeDtypeStruct((E, S), jnp.float32),
            jax.ShapeDtypeStruct((S, H), f16),
            jax.ShapeDtypeStruct((S, H), f16),
            jax.ShapeDtypeStruct((H, S), f16),
        ],
    )(x, gate_v_w, gate_o_w, wqk, acat, bqk, wv, bvs, rep)

    gate_sc = pl.kernel(
        _gate_coef_sc,
        out_type=jax.ShapeDtypeStruct((E, S), jnp.float32),
        scratch_types=[
            pltpu.VMEM((E, CH), jnp.float32),
            pltpu.VMEM((E, CH), jnp.float32),
        ],
        mesh=plsc.VectorSubcoreMesh(core_axis_name="c", subcore_axis_name="s"),
    )
    co = gate_sc(zo)

    nab = S // ABLK
    ctx_t = pl.pallas_call(
        _attn_kernel,
        grid=(NH // 2, nab),
        in_specs=[
            pl.BlockSpec((ABLK, 2 * DH), lambda h, s: (s, h)),
            pl.BlockSpec((S, 2 * DH), lambda h, s: (0, h)),
            pl.BlockSpec((2 * DH, S), lambda h, s: (h, 0)),
            pl.BlockSpec((S, 1), lambda h, s: (0, 0)),
        ],
        out_specs=pl.BlockSpec((2 * DH, ABLK), lambda h, s: (h, s)),
        out_shape=jax.ShapeDtypeStruct((H, S), f16),
    )(q, k, vt, mbias)

    out = pl.pallas_call(
        _omoe_kernel,
        grid=(nblk,),
        in_specs=[
            pl.BlockSpec((H, SBLK), lambda s: (0, s)),
            pl.BlockSpec((E, SBLK), lambda s: (0, s)),
            _full((E, H, H)), _full((H, E * R)), _full((E * R, H)),
            _full((E, E * R)),
        ],
        out_specs=pl.BlockSpec((SBLK, H), lambda s: (s, 0)),
        out_shape=jax.ShapeDtypeStruct((S, H), jnp.float32),
    )(ctx_t, co, wo, aocat, bos, rep)

    return out.reshape(B, S, H)
